# Initial kernel scaffold; baseline (speedup 1.0000x reference)
import functools

import jax
import jax.numpy as jnp
from jax import lax
from jax.experimental import pallas as pl
from jax.experimental.pallas import tpu as pltpu
from jax.experimental.pallas import tpu_sc as plsc

N = 2_000_000
INV_N_CHARGES = 1.0 / 20.0
NW = 32
L = 16
BLK = 4000
NBLK = N // BLK
BASE = NBLK // NW
EXTRA = NBLK % NW
VPB = BLK // L

_mesh = plsc.VectorSubcoreMesh(core_axis_name="c", subcore_axis_name="s")


def _tile_range(w):
    nblk = jnp.where(w < EXTRA, BASE + 1, BASE)
    bstart = jnp.where(w < EXTRA, w * (BASE + 1),
                       EXTRA * (BASE + 1) + (w - EXTRA) * BASE)
    return bstart, nblk


@functools.partial(
    pl.kernel,
    out_type=jax.ShapeDtypeStruct((NW, 3, L), jnp.float32),
    mesh=_mesh,
    scratch_types=[
        pltpu.VMEM((BLK,), jnp.float32),
        pltpu.VMEM((BLK,), jnp.float32),
        pltpu.VMEM((BLK,), jnp.int32),
        pltpu.VMEM((3, L), jnp.float32),
        pltpu.SemaphoreType.DMA,
        pltpu.SemaphoreType.DMA,
        pltpu.SemaphoreType.DMA,
    ],
)
def _reduce(x_hbm, c_hbm, i_hbm, out_hbm, xbuf, cbuf, ibuf, pbuf,
            semx, semc, semi):
    w = lax.axis_index("s") * 2 + lax.axis_index("c")
    bstart, nblk = _tile_range(w)
    zero = jnp.zeros((L,), jnp.float32)

    def blk_body(b, carry):
        s0, s1, cnt = carry
        base = pl.multiple_of((bstart + b) * BLK, BLK)
        cx = pltpu.async_copy(x_hbm.at[pl.ds(base, BLK)], xbuf, semx)
        cc = pltpu.async_copy(c_hbm.at[pl.ds(base, BLK)], cbuf, semc)
        ci = pltpu.async_copy(i_hbm.at[pl.ds(base, BLK)], ibuf, semi)
        cx.wait()
        cc.wait()
        ci.wait()

        def vec_body(j, carry2):
            s0, s1, cnt = carry2
            xv = xbuf[pl.ds(j * L, L)]
            cv = cbuf[pl.ds(j * L, L)]
            iv = ibuf[pl.ds(j * L, L)]
            p = xv * cv
            m0 = iv == 0
            s0 = s0 + jnp.where(m0, p, zero)
            s1 = s1 + jnp.where(m0, zero, p)
            cnt = cnt + jnp.where(m0, jnp.int32(1), jnp.int32(0))
            return s0, s1, cnt

        return lax.fori_loop(0, VPB, vec_body, (s0, s1, cnt))

    s0, s1, cnt = lax.fori_loop(0, nblk, blk_body,
                                (zero, zero, jnp.zeros((L,), jnp.int32)))
    pbuf[0, :] = s0
    pbuf[1, :] = s1
    pbuf[2, :] = plsc.bitcast(cnt, jnp.float32)
    pltpu.sync_copy(pbuf, out_hbm.at[w])


@functools.partial(
    pl.kernel,
    out_type=jax.ShapeDtypeStruct((N,), jnp.float32),
    mesh=_mesh,
    scratch_types=[
        pltpu.VMEM((BLK,), jnp.float32),
        pltpu.VMEM((BLK,), jnp.float32),
        pltpu.VMEM((BLK,), jnp.float32),
        pltpu.VMEM((NW, 3, L), jnp.float32),
        pltpu.SemaphoreType.DMA,
        pltpu.SemaphoreType.DMA,
        pltpu.SemaphoreType.DMA,
    ],
)
def _apply(x_hbm, c_hbm, p_hbm, out_hbm, xbuf, cbuf, obuf, pbuf,
           semx, semc, semo):
    w = lax.axis_index("s") * 2 + lax.axis_index("c")
    bstart, nblk = _tile_range(w)
    pltpu.sync_copy(p_hbm, pbuf)

    s0v = jnp.zeros((L,), jnp.float32)
    s1v = jnp.zeros((L,), jnp.float32)
    cntv = jnp.zeros((L,), jnp.int32)
    for t in range(NW):
        s0v = s0v + pbuf[t, 0, :]
        s1v = s1v + pbuf[t, 1, :]
        cntv = cntv + plsc.bitcast(pbuf[t, 2, :], jnp.int32)
    s0 = jnp.sum(s0v) * INV_N_CHARGES
    s1 = jnp.sum(s1v) * INV_N_CHARGES
    bnd = jnp.sum(cntv)
    iota = lax.iota(jnp.int32, (L,))

    def blk_body(b, _):
        base = pl.multiple_of((bstart + b) * BLK, BLK)
        cx = pltpu.async_copy(x_hbm.at[pl.ds(base, BLK)], xbuf, semx)
        cc = pltpu.async_copy(c_hbm.at[pl.ds(base, BLK)], cbuf, semc)
        cx.wait()
        cc.wait()

        def vec_body(j, _):
            xv = xbuf[pl.ds(j * L, L)]
            cv = cbuf[pl.ds(j * L, L)]
            pos = iota + (base + j * L)
            sel = jnp.where(pos < bnd, s0, s1)
            obuf[pl.ds(j * L, L)] = (xv - sel) * cv
            return 0

        lax.fori_loop(0, VPB, vec_body, 0)
        pltpu.async_copy(obuf, out_hbm.at[pl.ds(base, BLK)], semo).wait()
        return 0

    lax.fori_loop(0, nblk, blk_body, 0)


def kernel(x0, charges, chg_idx):
    idx32 = chg_idx.astype(jnp.int32)
    partials = _reduce(x0, charges, idx32)
    return _apply(x0, charges, partials)


# speedup vs baseline: 23.7603x; 23.7603x over previous
import functools

import jax
import jax.numpy as jnp
from jax import lax
from jax.experimental import pallas as pl
from jax.experimental.pallas import tpu as pltpu
from jax.experimental.pallas import tpu_sc as plsc

N = 2_000_000
INV_N_CHARGES = 1.0 / 20.0
NW = 32
L = 16
BLK = 4000
NBLK = N // BLK
BASE = NBLK // NW
EXTRA = NBLK % NW
VPB = BLK // L

_mesh = plsc.VectorSubcoreMesh(core_axis_name="c", subcore_axis_name="s")


def _tile_range(w):
    nblk = jnp.where(w < EXTRA, BASE + 1, BASE)
    bstart = jnp.where(w < EXTRA, w * (BASE + 1),
                       EXTRA * (BASE + 1) + (w - EXTRA) * BASE)
    return bstart, nblk


@functools.partial(
    pl.kernel,
    out_type=jax.ShapeDtypeStruct((NW, 3, L), jnp.float32),
    mesh=_mesh,
    scratch_types=[
        pltpu.VMEM((BLK,), jnp.float32),
        pltpu.VMEM((BLK,), jnp.float32),
        pltpu.VMEM((BLK,), jnp.int32),
        pltpu.VMEM((3, L), jnp.float32),
        pltpu.SemaphoreType.DMA,
        pltpu.SemaphoreType.DMA,
        pltpu.SemaphoreType.DMA,
    ],
)
def _reduce(x_hbm, c_hbm, i_hbm, out_hbm, xbuf, cbuf, ibuf, pbuf,
            semx, semc, semi):
    w = lax.axis_index("s") * 2 + lax.axis_index("c")
    bstart, nblk = _tile_range(w)
    zero = jnp.zeros((L,), jnp.float32)

    def blk_body(b, carry):
        s0, s1, cnt = carry
        base = pl.multiple_of((bstart + b) * BLK, BLK)
        cx = pltpu.async_copy(x_hbm.at[pl.ds(base, BLK)], xbuf, semx)
        cc = pltpu.async_copy(c_hbm.at[pl.ds(base, BLK)], cbuf, semc)
        ci = pltpu.async_copy(i_hbm.at[pl.ds(base, BLK)], ibuf, semi)
        cx.wait()
        cc.wait()
        ci.wait()

        def vec_body(j, carry2):
            s0, s1, cnt = carry2
            xv = xbuf[pl.ds(j * L, L)]
            cv = cbuf[pl.ds(j * L, L)]
            iv = ibuf[pl.ds(j * L, L)]
            p = xv * cv
            m0 = iv == 0
            s0 = s0 + jnp.where(m0, p, zero)
            s1 = s1 + jnp.where(m0, zero, p)
            cnt = cnt + jnp.where(m0, jnp.float32(1.0), jnp.float32(0.0))
            return s0, s1, cnt

        return lax.fori_loop(0, VPB, vec_body, (s0, s1, cnt))

    s0, s1, cnt = lax.fori_loop(0, nblk, blk_body, (zero, zero, zero))
    pbuf[0, :] = s0
    pbuf[1, :] = s1
    pbuf[2, :] = cnt
    pltpu.sync_copy(pbuf, out_hbm.at[w])


@functools.partial(
    pl.kernel,
    out_type=jax.ShapeDtypeStruct((N,), jnp.float32),
    mesh=_mesh,
    scratch_types=[
        pltpu.VMEM((BLK,), jnp.float32),
        pltpu.VMEM((BLK,), jnp.float32),
        pltpu.VMEM((BLK,), jnp.float32),
        pltpu.VMEM((NW, 3, L), jnp.float32),
        pltpu.VMEM((2 * L,), jnp.float32),
        pltpu.SemaphoreType.DMA,
        pltpu.SemaphoreType.DMA,
        pltpu.SemaphoreType.DMA,
    ],
)
def _apply(x_hbm, c_hbm, p_hbm, out_hbm, xbuf, cbuf, obuf, pbuf, rbuf,
           semx, semc, semo):
    w = lax.axis_index("s") * 2 + lax.axis_index("c")
    bstart, nblk = _tile_range(w)
    pltpu.sync_copy(p_hbm, pbuf)

    def all_lanes_sum(v):
        for k in (8, 4, 2, 1):
            rbuf[pl.ds(0, L)] = v
            rbuf[pl.ds(L, L)] = v
            v = v + rbuf[pl.ds(k, L)]
        return v

    s0v = jnp.zeros((L,), jnp.float32)
    s1v = jnp.zeros((L,), jnp.float32)
    cntv = jnp.zeros((L,), jnp.float32)
    for t in range(NW):
        s0v = s0v + pbuf[t, 0, :]
        s1v = s1v + pbuf[t, 1, :]
        cntv = cntv + pbuf[t, 2, :]
    s0 = all_lanes_sum(s0v) * INV_N_CHARGES
    s1 = all_lanes_sum(s1v) * INV_N_CHARGES
    bnd = all_lanes_sum(cntv).astype(jnp.int32)
    iota = lax.iota(jnp.int32, L)

    def blk_body(b, _):
        base = pl.multiple_of((bstart + b) * BLK, BLK)
        cx = pltpu.async_copy(x_hbm.at[pl.ds(base, BLK)], xbuf, semx)
        cc = pltpu.async_copy(c_hbm.at[pl.ds(base, BLK)], cbuf, semc)
        cx.wait()
        cc.wait()

        def vec_body(j, _):
            xv = xbuf[pl.ds(j * L, L)]
            cv = cbuf[pl.ds(j * L, L)]
            pos = iota + (base + j * L)
            sel = jnp.where(pos < bnd, s0, s1)
            obuf[pl.ds(j * L, L)] = (xv - sel) * cv
            return 0

        lax.fori_loop(0, VPB, vec_body, 0)
        pltpu.async_copy(obuf, out_hbm.at[pl.ds(base, BLK)], semo).wait()
        return 0

    lax.fori_loop(0, nblk, blk_body, 0)


def kernel(x0, charges, chg_idx):
    idx32 = chg_idx.astype(jnp.int32)
    partials = _reduce(x0, charges, idx32)
    return _apply(x0, charges, partials)


# speedup vs baseline: 31.8846x; 1.3419x over previous
import functools

import jax
import jax.numpy as jnp
from jax import lax
from jax.experimental import pallas as pl
from jax.experimental.pallas import tpu as pltpu
from jax.experimental.pallas import tpu_sc as plsc

N = 2_000_000
INV_N_CHARGES = 1.0 / 20.0
NW = 32
L = 16
BLK = 4000
NBLK = N // BLK
BASE = NBLK // NW
EXTRA = NBLK % NW
VPB = BLK // L

_mesh = plsc.VectorSubcoreMesh(core_axis_name="c", subcore_axis_name="s")


def _tile_range(w):
    nblk = jnp.where(w < EXTRA, BASE + 1, BASE)
    bstart = jnp.where(w < EXTRA, w * (BASE + 1),
                       EXTRA * (BASE + 1) + (w - EXTRA) * BASE)
    return bstart, nblk


@functools.partial(
    pl.kernel,
    out_type=jax.ShapeDtypeStruct((NW, 3, L), jnp.float32),
    mesh=_mesh,
    scratch_types=[
        pltpu.VMEM((BLK,), jnp.float32), pltpu.VMEM((BLK,), jnp.float32),
        pltpu.VMEM((BLK,), jnp.float32), pltpu.VMEM((BLK,), jnp.float32),
        pltpu.VMEM((BLK,), jnp.int32), pltpu.VMEM((BLK,), jnp.int32),
        pltpu.VMEM((3, L), jnp.float32),
        pltpu.SemaphoreType.DMA, pltpu.SemaphoreType.DMA,
        pltpu.SemaphoreType.DMA, pltpu.SemaphoreType.DMA,
        pltpu.SemaphoreType.DMA, pltpu.SemaphoreType.DMA,
    ],
)
def _reduce(x_hbm, c_hbm, i_hbm, out_hbm,
            xb0, xb1, cb0, cb1, ib0, ib1, pbuf,
            sx0, sx1, sc0, sc1, si0, si1):
    w = lax.axis_index("s") * 2 + lax.axis_index("c")
    bstart, nblk = _tile_range(w)
    zero = jnp.zeros((L,), jnp.float32)
    slots = ((xb0, cb0, ib0, sx0, sc0, si0),
             (xb1, cb1, ib1, sx1, sc1, si1))

    def issue(b, s):
        xb, cb, ib, sx, sc, si = slots[s]
        base = pl.multiple_of((bstart + b) * BLK, BLK)
        pltpu.async_copy(x_hbm.at[pl.ds(base, BLK)], xb, sx)
        pltpu.async_copy(c_hbm.at[pl.ds(base, BLK)], cb, sc)
        pltpu.async_copy(i_hbm.at[pl.ds(base, BLK)], ib, si)

    def wait_in(s):
        xb, cb, ib, sx, sc, si = slots[s]
        src = x_hbm.at[pl.ds(0, BLK)]
        pltpu.make_async_copy(src, xb, sx).wait()
        pltpu.make_async_copy(src, cb, sc).wait()
        isrc = i_hbm.at[pl.ds(0, BLK)]
        pltpu.make_async_copy(isrc, ib, si).wait()

    def acc_block(s):
        xb, cb, ib = slots[s][:3]
        carry = (pbuf[0, :], pbuf[1, :], pbuf[2, :])

        def vec_body(j, carry2):
            s0, s1, cnt = carry2
            xv = xb[pl.ds(j * L, L)]
            cv = cb[pl.ds(j * L, L)]
            iv = ib[pl.ds(j * L, L)]
            p = xv * cv
            m0 = iv == 0
            s0 = s0 + jnp.where(m0, p, zero)
            s1 = s1 + jnp.where(m0, zero, p)
            cnt = cnt + jnp.where(m0, jnp.float32(1.0), jnp.float32(0.0))
            return s0, s1, cnt

        s0, s1, cnt = lax.fori_loop(0, VPB, vec_body, carry, unroll=4)
        pbuf[0, :] = s0
        pbuf[1, :] = s1
        pbuf[2, :] = cnt

    pbuf[0, :] = zero
    pbuf[1, :] = zero
    pbuf[2, :] = zero
    issue(0, 0)
    issue(1, 1)

    def pair_body(pr, _):
        b = 2 * pr
        wait_in(0)
        acc_block(0)

        @pl.when(b + 2 < nblk)
        def _():
            issue(b + 2, 0)

        wait_in(1)
        acc_block(1)

        @pl.when(b + 3 < nblk)
        def _():
            issue(b + 3, 1)

        return 0

    lax.fori_loop(0, nblk // 2, pair_body, 0)

    @pl.when(lax.rem(nblk, 2) == 1)
    def _():
        wait_in(0)
        acc_block(0)

    pltpu.sync_copy(pbuf, out_hbm.at[w])


@functools.partial(
    pl.kernel,
    out_type=jax.ShapeDtypeStruct((N,), jnp.float32),
    mesh=_mesh,
    scratch_types=[
        pltpu.VMEM((BLK,), jnp.float32), pltpu.VMEM((BLK,), jnp.float32),
        pltpu.VMEM((BLK,), jnp.float32), pltpu.VMEM((BLK,), jnp.float32),
        pltpu.VMEM((BLK,), jnp.float32), pltpu.VMEM((BLK,), jnp.float32),
        pltpu.VMEM((NW, 3, L), jnp.float32),
        pltpu.VMEM((2 * L,), jnp.float32),
        pltpu.SemaphoreType.DMA, pltpu.SemaphoreType.DMA,
        pltpu.SemaphoreType.DMA, pltpu.SemaphoreType.DMA,
        pltpu.SemaphoreType.DMA, pltpu.SemaphoreType.DMA,
    ],
)
def _apply(x_hbm, c_hbm, p_hbm, out_hbm,
           xb0, xb1, cb0, cb1, ob0, ob1, pbuf, rbuf,
           sx0, sx1, sc0, sc1, so0, so1):
    w = lax.axis_index("s") * 2 + lax.axis_index("c")
    bstart, nblk = _tile_range(w)
    pltpu.sync_copy(p_hbm, pbuf)

    def all_lanes_sum(v):
        for k in (8, 4, 2, 1):
            rbuf[pl.ds(0, L)] = v
            rbuf[pl.ds(L, L)] = v
            v = v + rbuf[pl.ds(k, L)]
        return v

    s0v = jnp.zeros((L,), jnp.float32)
    s1v = jnp.zeros((L,), jnp.float32)
    cntv = jnp.zeros((L,), jnp.float32)
    for t in range(NW):
        s0v = s0v + pbuf[t, 0, :]
        s1v = s1v + pbuf[t, 1, :]
        cntv = cntv + pbuf[t, 2, :]
    s0 = all_lanes_sum(s0v) * INV_N_CHARGES
    s1 = all_lanes_sum(s1v) * INV_N_CHARGES
    bnd = all_lanes_sum(cntv).astype(jnp.int32)
    iota = lax.iota(jnp.int32, L)

    slots = ((xb0, cb0, ob0, sx0, sc0, so0),
             (xb1, cb1, ob1, sx1, sc1, so1))

    def issue(b, s):
        xb, cb = slots[s][:2]
        sx, sc = slots[s][3:5]
        base = pl.multiple_of((bstart + b) * BLK, BLK)
        pltpu.async_copy(x_hbm.at[pl.ds(base, BLK)], xb, sx)
        pltpu.async_copy(c_hbm.at[pl.ds(base, BLK)], cb, sc)

    def wait_in(s):
        xb, cb = slots[s][:2]
        sx, sc = slots[s][3:5]
        src = x_hbm.at[pl.ds(0, BLK)]
        pltpu.make_async_copy(src, xb, sx).wait()
        pltpu.make_async_copy(src, cb, sc).wait()

    def wait_scatter(s):
        ob, so = slots[s][2], slots[s][5]
        pltpu.make_async_copy(ob, out_hbm.at[pl.ds(0, BLK)], so).wait()

    def process(b, s):
        xb, cb, ob = slots[s][:3]
        so = slots[s][5]
        base = pl.multiple_of((bstart + b) * BLK, BLK)

        def vec_body(j, _):
            xv = xb[pl.ds(j * L, L)]
            cv = cb[pl.ds(j * L, L)]
            pos = iota + (base + j * L)
            sel = jnp.where(pos < bnd, s0, s1)
            ob[pl.ds(j * L, L)] = (xv - sel) * cv
            return 0

        lax.fori_loop(0, VPB, vec_body, 0, unroll=4)
        pltpu.async_copy(ob, out_hbm.at[pl.ds(base, BLK)], so)

    issue(0, 0)
    issue(1, 1)

    def pair_body(pr, _):
        b = 2 * pr
        wait_in(0)

        @pl.when(pr > 0)
        def _():
            wait_scatter(0)

        process(b, 0)

        @pl.when(b + 2 < nblk)
        def _():
            issue(b + 2, 0)

        wait_in(1)

        @pl.when(pr > 0)
        def _():
            wait_scatter(1)

        process(b + 1, 1)

        @pl.when(b + 3 < nblk)
        def _():
            issue(b + 3, 1)

        return 0

    lax.fori_loop(0, nblk // 2, pair_body, 0)

    @pl.when(lax.rem(nblk, 2) == 1)
    def _():
        wait_in(0)
        wait_scatter(0)
        process(nblk - 1, 0)

    wait_scatter(0)
    wait_scatter(1)


def kernel(x0, charges, chg_idx):
    idx32 = chg_idx.astype(jnp.int32)
    partials = _reduce(x0, charges, idx32)
    return _apply(x0, charges, partials)


# speedup vs baseline: 35.9023x; 1.1260x over previous
import functools

import jax
import jax.numpy as jnp
from jax import lax
from jax.experimental import pallas as pl
from jax.experimental.pallas import tpu as pltpu
from jax.experimental.pallas import tpu_sc as plsc

N = 2_000_000
INV_N_CHARGES = 1.0 / 20.0
NW = 32
L = 16
BLK = 4000
NBLK = N // BLK
BASE = NBLK // NW
EXTRA = NBLK % NW
VPB = BLK // L

_mesh = plsc.VectorSubcoreMesh(core_axis_name="c", subcore_axis_name="s")


def _tile_range(w):
    nblk = jnp.where(w < EXTRA, BASE + 1, BASE)
    bstart = jnp.where(w < EXTRA, w * (BASE + 1),
                       EXTRA * (BASE + 1) + (w - EXTRA) * BASE)
    return bstart, nblk


@functools.partial(
    pl.kernel,
    out_type=jax.ShapeDtypeStruct((NW, 3, L), jnp.float32),
    mesh=_mesh,
    scratch_types=[
        pltpu.VMEM((BLK,), jnp.float32), pltpu.VMEM((BLK,), jnp.float32),
        pltpu.VMEM((BLK,), jnp.float32), pltpu.VMEM((BLK,), jnp.float32),
        pltpu.VMEM((BLK,), jnp.int32), pltpu.VMEM((BLK,), jnp.int32),
        pltpu.VMEM((3, L), jnp.float32),
        pltpu.SemaphoreType.DMA, pltpu.SemaphoreType.DMA,
        pltpu.SemaphoreType.DMA, pltpu.SemaphoreType.DMA,
        pltpu.SemaphoreType.DMA, pltpu.SemaphoreType.DMA,
    ],
)
def _reduce(x_hbm, c_hbm, i_hbm, out_hbm,
            xb0, xb1, cb0, cb1, ib0, ib1, pbuf,
            sx0, sx1, sc0, sc1, si0, si1):
    w = lax.axis_index("s") * 2 + lax.axis_index("c")
    bstart, nblk = _tile_range(w)
    zero = jnp.zeros((L,), jnp.float32)
    slots = ((xb0, cb0, ib0, sx0, sc0, si0),
             (xb1, cb1, ib1, sx1, sc1, si1))

    def issue(b, s):
        xb, cb, ib, sx, sc, si = slots[s]
        base = pl.multiple_of((bstart + b) * BLK, BLK)
        pltpu.async_copy(x_hbm.at[pl.ds(base, BLK)], xb, sx)
        pltpu.async_copy(c_hbm.at[pl.ds(base, BLK)], cb, sc)
        pltpu.async_copy(i_hbm.at[pl.ds(base, BLK)], ib, si)

    def wait_in(s):
        xb, cb, ib, sx, sc, si = slots[s]
        src = x_hbm.at[pl.ds(0, BLK)]
        pltpu.make_async_copy(src, xb, sx).wait()
        pltpu.make_async_copy(src, cb, sc).wait()
        isrc = i_hbm.at[pl.ds(0, BLK)]
        pltpu.make_async_copy(isrc, ib, si).wait()

    def acc_block(s):
        xb, cb, ib = slots[s][:3]
        carry = (pbuf[0, :], pbuf[1, :], pbuf[2, :])

        def vec_body(i, carry2):
            s0, s1, cnt = carry2
            xv = xb[pl.ds(i, L)]
            cv = cb[pl.ds(i, L)]
            iv = ib[pl.ds(i, L)]
            p = xv * cv
            m0 = iv == 0
            s0 = s0 + jnp.where(m0, p, zero)
            s1 = s1 + jnp.where(m0, zero, p)
            cnt = cnt + jnp.where(m0, jnp.float32(1.0), jnp.float32(0.0))
            return s0, s1, cnt

        s0, s1, cnt = plsc.parallel_loop(0, BLK, L, unroll=4,
                                         carry=carry)(vec_body)
        pbuf[0, :] = s0
        pbuf[1, :] = s1
        pbuf[2, :] = cnt

    pbuf[0, :] = zero
    pbuf[1, :] = zero
    pbuf[2, :] = zero
    issue(0, 0)
    issue(1, 1)

    def pair_body(pr, _):
        b = 2 * pr
        wait_in(0)
        acc_block(0)

        @pl.when(b + 2 < nblk)
        def _():
            issue(b + 2, 0)

        wait_in(1)
        acc_block(1)

        @pl.when(b + 3 < nblk)
        def _():
            issue(b + 3, 1)

        return 0

    lax.fori_loop(0, nblk // 2, pair_body, 0)

    @pl.when(lax.rem(nblk, 2) == 1)
    def _():
        wait_in(0)
        acc_block(0)

    pltpu.sync_copy(pbuf, out_hbm.at[w])


@functools.partial(
    pl.kernel,
    out_type=jax.ShapeDtypeStruct((N,), jnp.float32),
    mesh=_mesh,
    scratch_types=[
        pltpu.VMEM((BLK,), jnp.float32), pltpu.VMEM((BLK,), jnp.float32),
        pltpu.VMEM((BLK,), jnp.float32), pltpu.VMEM((BLK,), jnp.float32),
        pltpu.VMEM((BLK,), jnp.float32), pltpu.VMEM((BLK,), jnp.float32),
        pltpu.VMEM((NW, 3, L), jnp.float32),
        pltpu.VMEM((2 * L,), jnp.float32),
        pltpu.SemaphoreType.DMA, pltpu.SemaphoreType.DMA,
        pltpu.SemaphoreType.DMA, pltpu.SemaphoreType.DMA,
        pltpu.SemaphoreType.DMA, pltpu.SemaphoreType.DMA,
    ],
)
def _apply(x_hbm, c_hbm, p_hbm, out_hbm,
           xb0, xb1, cb0, cb1, ob0, ob1, pbuf, rbuf,
           sx0, sx1, sc0, sc1, so0, so1):
    w = lax.axis_index("s") * 2 + lax.axis_index("c")
    bstart, nblk = _tile_range(w)
    pltpu.sync_copy(p_hbm, pbuf)

    def all_lanes_sum(v):
        for k in (8, 4, 2, 1):
            rbuf[pl.ds(0, L)] = v
            rbuf[pl.ds(L, L)] = v
            v = v + rbuf[pl.ds(k, L)]
        return v

    s0v = jnp.zeros((L,), jnp.float32)
    s1v = jnp.zeros((L,), jnp.float32)
    cntv = jnp.zeros((L,), jnp.float32)
    for t in range(NW):
        s0v = s0v + pbuf[t, 0, :]
        s1v = s1v + pbuf[t, 1, :]
        cntv = cntv + pbuf[t, 2, :]
    s0 = all_lanes_sum(s0v) * INV_N_CHARGES
    s1 = all_lanes_sum(s1v) * INV_N_CHARGES
    bndf = all_lanes_sum(cntv)
    bnd = bndf.astype(jnp.int32)
    bnd_s = bndf[0].astype(jnp.int32)
    iota = lax.iota(jnp.int32, L)

    slots = ((xb0, cb0, ob0, sx0, sc0, so0),
             (xb1, cb1, ob1, sx1, sc1, so1))

    def issue(b, s):
        xb, cb = slots[s][:2]
        sx, sc = slots[s][3:5]
        base = pl.multiple_of((bstart + b) * BLK, BLK)
        pltpu.async_copy(x_hbm.at[pl.ds(base, BLK)], xb, sx)
        pltpu.async_copy(c_hbm.at[pl.ds(base, BLK)], cb, sc)

    def wait_in(s):
        xb, cb = slots[s][:2]
        sx, sc = slots[s][3:5]
        src = x_hbm.at[pl.ds(0, BLK)]
        pltpu.make_async_copy(src, xb, sx).wait()
        pltpu.make_async_copy(src, cb, sc).wait()

    def wait_scatter(s):
        ob, so = slots[s][2], slots[s][5]
        pltpu.make_async_copy(ob, out_hbm.at[pl.ds(0, BLK)], so).wait()

    def process(b, s):
        xb, cb, ob = slots[s][:3]
        so = slots[s][5]
        base = pl.multiple_of((bstart + b) * BLK, BLK)

        def const_body(segv):
            def vec_body(i):
                xv = xb[pl.ds(i, L)]
                cv = cb[pl.ds(i, L)]
                ob[pl.ds(i, L)] = (xv - segv) * cv
            return vec_body

        def sel_body(i):
            xv = xb[pl.ds(i, L)]
            cv = cb[pl.ds(i, L)]
            pos = iota + (base + i)
            sel = jnp.where(pos < bnd, s0, s1)
            ob[pl.ds(i, L)] = (xv - sel) * cv

        below = base + BLK <= bnd_s
        above = base >= bnd_s

        @pl.when(below)
        def _():
            plsc.parallel_loop(0, BLK, L, unroll=4)(const_body(s0))

        @pl.when(above)
        def _():
            plsc.parallel_loop(0, BLK, L, unroll=4)(const_body(s1))

        @pl.when(jnp.logical_not(jnp.logical_or(below, above)))
        def _():
            plsc.parallel_loop(0, BLK, L, unroll=4)(sel_body)

        pltpu.async_copy(ob, out_hbm.at[pl.ds(base, BLK)], so)

    issue(0, 0)
    issue(1, 1)

    def pair_body(pr, _):
        b = 2 * pr
        wait_in(0)

        @pl.when(pr > 0)
        def _():
            wait_scatter(0)

        process(b, 0)

        @pl.when(b + 2 < nblk)
        def _():
            issue(b + 2, 0)

        wait_in(1)

        @pl.when(pr > 0)
        def _():
            wait_scatter(1)

        process(b + 1, 1)

        @pl.when(b + 3 < nblk)
        def _():
            issue(b + 3, 1)

        return 0

    lax.fori_loop(0, nblk // 2, pair_body, 0)

    @pl.when(lax.rem(nblk, 2) == 1)
    def _():
        wait_in(0)
        wait_scatter(0)
        process(nblk - 1, 0)

    wait_scatter(0)
    wait_scatter(1)


def kernel(x0, charges, chg_idx):
    idx32 = chg_idx.astype(jnp.int32)
    partials = _reduce(x0, charges, idx32)
    return _apply(x0, charges, partials)
